# baseline (speedup 1.0000x reference)
import functools

import jax
import jax.numpy as jnp
from jax import lax
from jax.experimental import pallas as pl
from jax.experimental.pallas import tpu as pltpu
from jax.experimental.pallas import tpu_sc as plsc

N = 10000
E = 160000
NW = 32
NSUB = 16
EPW = E // NW
K = 128
CH = (EPW + K - 1) // K
EPW_PAD = CH * K
NPAD = 10240
ROWS_PER_SUB = NPAD // NSUB
TRASH = NPAD - 8

BM = 512
BN = 1024


def _pad_idx(idx, fill):
    r = idx.reshape(NW, EPW)
    pad = jnp.full((NW, EPW_PAD - EPW), fill, dtype=idx.dtype)
    return jnp.concatenate([r, pad], axis=1).reshape(NW, CH, K).astype(jnp.int32)


def _agg_body(y_hbm, isrc_hbm, idst_hbm, out_hbm,
              isrc_v, idst_v, rows_v, zbuf_v, acc_sh, sem_g, sem_s):
    cid = lax.axis_index("c")
    sid = lax.axis_index("s")
    w = cid * NSUB + sid
    D = zbuf_v.shape[1]

    for r in range(16):
        for c in range(D // 16):
            zbuf_v[r, pl.ds(c * 16, 16)] = jnp.zeros((16,), jnp.float32)
    zd = []
    for t in range(ROWS_PER_SUB // 16):
        zd.append(pltpu.async_copy(
            zbuf_v, acc_sh.at[pl.ds(sid * ROWS_PER_SUB + t * 16, 16)], sem_g))
    for d in zd:
        d.wait()

    pltpu.sync_copy(isrc_hbm.at[w], isrc_v)
    pltpu.sync_copy(idst_hbm.at[w], idst_v)
    plsc.subcore_barrier()

    NB = 2

    def fire_gather(j, b):
        pltpu.async_copy(y_hbm.at[isrc_v.at[j]], rows_v.at[b], sem_g)

    def wait_gather(j, b):
        pltpu.make_async_copy(y_hbm.at[isrc_v.at[j]], rows_v.at[b], sem_g).wait()

    def fire_scatter(j, b):
        pltpu.async_copy(rows_v.at[b], acc_sh.at[idst_v.at[j]], sem_s, add=True)

    def wait_scatter(j, b):
        pltpu.make_async_copy(rows_v.at[b], acc_sh.at[idst_v.at[j]], sem_s).wait()

    for j0 in range(NB - 1):
        fire_gather(j0, j0)

    def chunk(j, k):
        wait_gather(j, k)
        fire_scatter(j, k)

        @pl.when(j >= 1)
        def _():
            wait_scatter(j - 1, (k - 1) % NB)

        @pl.when(j + NB - 1 < CH)
        def _():
            fire_gather(j + NB - 1, (k - 1) % NB)

    def body(it, _):
        for k in range(NB):
            chunk(NB * it + k, k)
        return 0

    lax.fori_loop(0, CH // NB, body, 0)
    wait_scatter(CH - 1, (CH - 1) % NB)
    plsc.subcore_barrier()

    pltpu.sync_copy(acc_sh.at[pl.ds(sid * ROWS_PER_SUB, ROWS_PER_SUB)],
                    out_hbm.at[cid].at[pl.ds(sid * ROWS_PER_SUB, ROWS_PER_SUB)])


def _sc_agg(y, isrc, idst, D):
    mesh = plsc.VectorSubcoreMesh(core_axis_name="c", subcore_axis_name="s")
    return pl.kernel(
        _agg_body,
        out_type=jax.ShapeDtypeStruct((2, NPAD, D), jnp.float32),
        mesh=mesh,
        scratch_types=[
            pltpu.VMEM((CH, K), jnp.int32),
            pltpu.VMEM((CH, K), jnp.int32),
            pltpu.VMEM((2, K, D), jnp.float32),
            pltpu.VMEM((16, D), jnp.float32),
            pltpu.VMEM_SHARED((NPAD, D), jnp.float32),
            pltpu.SemaphoreType.DMA,
            pltpu.SemaphoreType.DMA,
        ],
    )(y, isrc, idst)


def _deg_body(idst_hbm, out_hbm, idst_v, ones_v, zbuf_v, acc_sh, sem_g):
    cid = lax.axis_index("c")
    sid = lax.axis_index("s")
    w = cid * NSUB + sid

    for c in range(8):
        zbuf_v[pl.ds(c * 16, 16)] = jnp.zeros((16,), jnp.float32)
        ones_v[pl.ds(c * 16, 16)] = jnp.ones((16,), jnp.float32)
    zd = []
    for t in range(ROWS_PER_SUB // 128):
        zd.append(pltpu.async_copy(
            zbuf_v, acc_sh.at[pl.ds(sid * ROWS_PER_SUB + t * 128, 128)], sem_g))
    for d in zd:
        d.wait()

    pltpu.sync_copy(idst_hbm.at[w], idst_v)
    plsc.subcore_barrier()

    def body(j, _):
        pltpu.sync_copy(ones_v, acc_sh.at[idst_v.at[j]], add=True)
        return 0

    lax.fori_loop(0, CH, body, 0)
    plsc.subcore_barrier()

    pltpu.sync_copy(acc_sh.at[pl.ds(sid * ROWS_PER_SUB, ROWS_PER_SUB)],
                    out_hbm.at[cid].at[pl.ds(sid * ROWS_PER_SUB, ROWS_PER_SUB)])


def _sc_deg(idst):
    mesh = plsc.VectorSubcoreMesh(core_axis_name="c", subcore_axis_name="s")
    return pl.kernel(
        _deg_body,
        out_type=jax.ShapeDtypeStruct((2, NPAD), jnp.float32),
        mesh=mesh,
        scratch_types=[
            pltpu.VMEM((CH, K), jnp.int32),
            pltpu.VMEM((K,), jnp.float32),
            pltpu.VMEM((128,), jnp.float32),
            pltpu.VMEM_SHARED((NPAD,), jnp.float32),
            pltpu.SemaphoreType.DMA,
        ],
    )(idst)


BR = 512
GRID_R = NPAD // BR


def _y1_body(degp_ref, x_ref, wg1_ref, y1_ref, dinv_ref):
    deg = degp_ref[0] + degp_ref[1] + 1.0
    dinv = lax.rsqrt(deg)
    dinv_ref[...] = dinv[:, None]
    y1_ref[...] = lax.dot_general(
        x_ref[...], wg1_ref[...], (((1,), (0,)), ((), ())),
        preferred_element_type=jnp.float32) * dinv[:, None]


def _y1_kernel(degp, x, W_g1):
    return pl.pallas_call(
        _y1_body,
        grid=(GRID_R,),
        in_specs=[
            pl.BlockSpec((2, BR), lambda i: (0, i)),
            pl.BlockSpec((BR, 128), lambda i: (i, 0)),
            pl.BlockSpec((128, 128), lambda i: (0, 0)),
        ],
        out_specs=[
            pl.BlockSpec((BR, 128), lambda i: (i, 0)),
            pl.BlockSpec((BR, 1), lambda i: (i, 0)),
        ],
        out_shape=[
            jax.ShapeDtypeStruct((N, 128), jnp.float32),
            jax.ShapeDtypeStruct((N, 1), jnp.float32),
        ],
    )(degp, x, W_g1)


def _y2_body(aggp_ref, y1_ref, dinv_ref, bg1_ref, wg2_ref, y2p_ref):
    agg = aggp_ref[0] + aggp_ref[1] + y1_ref[...]
    z1 = jax.nn.relu(agg * dinv_ref[...] + bg1_ref[...])
    y2 = lax.dot_general(z1, wg2_ref[...], (((1,), (0,)), ((), ())),
                         preferred_element_type=jnp.float32) * dinv_ref[...]
    y2p_ref[:, :64] = y2
    y2p_ref[:, 64:] = jnp.zeros_like(y2)


def _y2_kernel(aggp1, y1, dinv, b_g1, W_g2):
    return pl.pallas_call(
        _y2_body,
        grid=(GRID_R,),
        in_specs=[
            pl.BlockSpec((2, BR, 128), lambda i: (0, i, 0)),
            pl.BlockSpec((BR, 128), lambda i: (i, 0)),
            pl.BlockSpec((BR, 1), lambda i: (i, 0)),
            pl.BlockSpec((1, 128), lambda i: (0, 0)),
            pl.BlockSpec((128, 64), lambda i: (0, 0)),
        ],
        out_specs=pl.BlockSpec((BR, 128), lambda i: (i, 0)),
        out_shape=jax.ShapeDtypeStruct((NPAD, 128), jnp.float32),
    )(aggp1, y1, dinv, b_g1.reshape(1, 128), W_g2)


def _dec_body(aggp_ref, y2p_ref, dinv_ref, batch_ref, bg2_ref,
              wd1_ref, bd1_ref, wd2_ref, bd2_ref, we_ref, be_ref,
              z_ref, xr_ref, el_ref, zg_ref):
    i = pl.program_id(0)
    y2 = y2p_ref[:, :64]
    agg = aggp_ref[0, :, :64] + aggp_ref[1, :, :64] + y2
    z = jax.nn.relu(agg * dinv_ref[...] + bg2_ref[...])
    z_ref[...] = z
    h = jax.nn.relu(lax.dot_general(z, wd1_ref[...], (((1,), (0,)), ((), ())),
                                    preferred_element_type=jnp.float32)
                    + bd1_ref[...])
    xr_ref[...] = lax.dot_general(h, wd2_ref[...], (((1,), (0,)), ((), ())),
                                  preferred_element_type=jnp.float32) + bd2_ref[...]
    el_ref[...] = lax.dot_general(z, we_ref[...], (((1,), (0,)), ((), ())),
                                  preferred_element_type=jnp.float32) + be_ref[...]

    @pl.when(i == 0)
    def _():
        zg_ref[...] = jnp.full_like(zg_ref, -jnp.inf)

    row = i * BR + lax.broadcasted_iota(jnp.int32, (BR, 1), 0)
    valid = row < N
    bvec = batch_ref[...]

    lv = jnp.minimum(BR - 1, N - 1 - i * BR)
    b0 = jnp.clip(bvec[0, 0], 0, 63)
    b1 = jnp.clip(batch_ref[pl.ds(lv, 1), :][0, 0], b0, 63)

    def seg_body(s, _):
        mask = (bvec == s) & valid
        vals = jnp.where(mask, z, -jnp.inf)
        m = jnp.max(vals, axis=0, keepdims=True)
        zg_ref[pl.ds(s, 1), :] = jnp.maximum(zg_ref[pl.ds(s, 1), :], m)
        return 0

    lax.fori_loop(b0, b1 + 1, seg_body, 0)


def _dec_kernel(aggp2, y2p, dinv, batch, b_g2, W_d1, b_d1, W_d2, b_d2, W_e, b_e):
    return pl.pallas_call(
        _dec_body,
        grid=(GRID_R,),
        in_specs=[
            pl.BlockSpec((2, BR, 128), lambda i: (0, i, 0)),
            pl.BlockSpec((BR, 128), lambda i: (i, 0)),
            pl.BlockSpec((BR, 1), lambda i: (i, 0)),
            pl.BlockSpec((BR, 1), lambda i: (i, 0)),
            pl.BlockSpec((1, 64), lambda i: (0, 0)),
            pl.BlockSpec((64, 128), lambda i: (0, 0)),
            pl.BlockSpec((1, 128), lambda i: (0, 0)),
            pl.BlockSpec((128, 128), lambda i: (0, 0)),
            pl.BlockSpec((1, 128), lambda i: (0, 0)),
            pl.BlockSpec((64, 64), lambda i: (0, 0)),
            pl.BlockSpec((1, 64), lambda i: (0, 0)),
        ],
        out_specs=[
            pl.BlockSpec((BR, 64), lambda i: (i, 0)),
            pl.BlockSpec((BR, 128), lambda i: (i, 0)),
            pl.BlockSpec((BR, 64), lambda i: (i, 0)),
            pl.BlockSpec((64, 64), lambda i: (0, 0)),
        ],
        out_shape=[
            jax.ShapeDtypeStruct((N, 64), jnp.float32),
            jax.ShapeDtypeStruct((N, 128), jnp.float32),
            jax.ShapeDtypeStruct((N, 64), jnp.float32),
            jax.ShapeDtypeStruct((64, 64), jnp.float32),
        ],
    )(aggp2, y2p, dinv, batch, b_g2.reshape(1, 64), W_d1, b_d1.reshape(1, 128),
      W_d2, b_d2.reshape(1, 128), W_e, b_e.reshape(1, 64))


def _adj_body(l_ref, lt_ref, out_ref):
    acc = lax.dot_general(l_ref[...], lt_ref[...], (((1,), (0,)), ((), ())),
                          preferred_element_type=jnp.float32)
    out_ref[...] = jax.nn.sigmoid(acc)


def _adj_recon(el):
    elt = el.T
    n = el.shape[0]
    grid = (pl.cdiv(n, BM), pl.cdiv(n, BN))
    return pl.pallas_call(
        _adj_body,
        grid=grid,
        in_specs=[
            pl.BlockSpec((BM, el.shape[1]), lambda i, j: (i, 0)),
            pl.BlockSpec((el.shape[1], BN), lambda i, j: (0, j)),
        ],
        out_specs=pl.BlockSpec((BM, BN), lambda i, j: (i, j)),
        out_shape=jax.ShapeDtypeStruct((n, n), jnp.float32),
    )(el, elt)


def kernel(x, edge_index, batch, W_g1, b_g1, W_g2, b_g2, W_d1, b_d1, W_d2, b_d2, W_e, b_e):
    src = edge_index[0].astype(jnp.int32)
    dst = edge_index[1].astype(jnp.int32)
    isrc = _pad_idx(src, 0)
    idst = _pad_idx(dst, TRASH)

    degp = _sc_deg(idst)
    y1, dinv = _y1_kernel(degp, x, W_g1)
    aggp1 = _sc_agg(y1, isrc, idst, 128)
    y2p = _y2_kernel(aggp1, y1, dinv, b_g1, W_g2)
    aggp2 = _sc_agg(y2p, isrc, idst, 128)
    z, x_recon, edge_logits, z_g = _dec_kernel(
        aggp2, y2p, dinv, batch.astype(jnp.int32).reshape(N, 1),
        b_g2, W_d1, b_d1, W_d2, b_d2, W_e, b_e)
    adj_recon = _adj_recon(edge_logits)
    return (z, z_g, x_recon, adj_recon)


# speedup vs baseline: 1.1402x; 1.1402x over previous
import functools

import jax
import jax.numpy as jnp
from jax import lax
from jax.experimental import pallas as pl
from jax.experimental.pallas import tpu as pltpu
from jax.experimental.pallas import tpu_sc as plsc

N = 10000
E = 160000
NW = 32
NSUB = 16
EPW = E // NW
K = 128
CH = (EPW + K - 1) // K
EPW_PAD = CH * K
NPAD = 10240
ROWS_PER_SUB = NPAD // NSUB
TRASH = NPAD - 8

BM = 2048
BN = 1024


def _pad_idx(idx, fill):
    r = idx.reshape(NW, EPW)
    pad = jnp.full((NW, EPW_PAD - EPW), fill, dtype=idx.dtype)
    return jnp.concatenate([r, pad], axis=1).reshape(NW, CH, K).astype(jnp.int32)


def _agg_body(y_hbm, isrc_hbm, idst_hbm, out_hbm,
              isrc_v, idst_v, rows_v, zbuf_v, acc_sh, sem_g, sem_s):
    cid = lax.axis_index("c")
    sid = lax.axis_index("s")
    w = cid * NSUB + sid
    D = zbuf_v.shape[1]

    for r in range(16):
        for c in range(D // 16):
            zbuf_v[r, pl.ds(c * 16, 16)] = jnp.zeros((16,), jnp.float32)
    zd = []
    for t in range(ROWS_PER_SUB // 16):
        zd.append(pltpu.async_copy(
            zbuf_v, acc_sh.at[pl.ds(sid * ROWS_PER_SUB + t * 16, 16)], sem_g))
    for d in zd:
        d.wait()

    pltpu.sync_copy(isrc_hbm.at[w], isrc_v)
    pltpu.sync_copy(idst_hbm.at[w], idst_v)
    plsc.subcore_barrier()

    NB = 2

    def fire_gather(j, b):
        pltpu.async_copy(y_hbm.at[isrc_v.at[j]], rows_v.at[b], sem_g)

    def wait_gather(j, b):
        pltpu.make_async_copy(y_hbm.at[isrc_v.at[j]], rows_v.at[b], sem_g).wait()

    def fire_scatter(j, b):
        pltpu.async_copy(rows_v.at[b], acc_sh.at[idst_v.at[j]], sem_s, add=True)

    def wait_scatter(j, b):
        pltpu.make_async_copy(rows_v.at[b], acc_sh.at[idst_v.at[j]], sem_s).wait()

    for j0 in range(NB - 1):
        fire_gather(j0, j0)

    def chunk(j, k):
        wait_gather(j, k)
        fire_scatter(j, k)

        @pl.when(j >= 1)
        def _():
            wait_scatter(j - 1, (k - 1) % NB)

        @pl.when(j + NB - 1 < CH)
        def _():
            fire_gather(j + NB - 1, (k - 1) % NB)

    def body(it, _):
        for k in range(NB):
            chunk(NB * it + k, k)
        return 0

    lax.fori_loop(0, CH // NB, body, 0)
    wait_scatter(CH - 1, (CH - 1) % NB)
    plsc.subcore_barrier()

    pltpu.sync_copy(acc_sh.at[pl.ds(sid * ROWS_PER_SUB, ROWS_PER_SUB)],
                    out_hbm.at[cid].at[pl.ds(sid * ROWS_PER_SUB, ROWS_PER_SUB)])


def _sc_agg(y, isrc, idst, D):
    mesh = plsc.VectorSubcoreMesh(core_axis_name="c", subcore_axis_name="s")
    return pl.kernel(
        _agg_body,
        out_type=jax.ShapeDtypeStruct((2, NPAD, D), jnp.float32),
        mesh=mesh,
        scratch_types=[
            pltpu.VMEM((CH, K), jnp.int32),
            pltpu.VMEM((CH, K), jnp.int32),
            pltpu.VMEM((2, K, D), jnp.float32),
            pltpu.VMEM((16, D), jnp.float32),
            pltpu.VMEM_SHARED((NPAD, D), jnp.float32),
            pltpu.SemaphoreType.DMA,
            pltpu.SemaphoreType.DMA,
        ],
    )(y, isrc, idst)


def _deg_body(idst_hbm, out_hbm, idst_v, ones_v, zbuf_v, acc_sh, sem_g):
    cid = lax.axis_index("c")
    sid = lax.axis_index("s")
    w = cid * NSUB + sid

    for c in range(8):
        zbuf_v[pl.ds(c * 16, 16)] = jnp.zeros((16,), jnp.float32)
        ones_v[pl.ds(c * 16, 16)] = jnp.ones((16,), jnp.float32)
    zd = []
    for t in range(ROWS_PER_SUB // 128):
        zd.append(pltpu.async_copy(
            zbuf_v, acc_sh.at[pl.ds(sid * ROWS_PER_SUB + t * 128, 128)], sem_g))
    for d in zd:
        d.wait()

    pltpu.sync_copy(idst_hbm.at[w], idst_v)
    plsc.subcore_barrier()

    def body(j, _):
        pltpu.sync_copy(ones_v, acc_sh.at[idst_v.at[j]], add=True)
        return 0

    lax.fori_loop(0, CH, body, 0)
    plsc.subcore_barrier()

    pltpu.sync_copy(acc_sh.at[pl.ds(sid * ROWS_PER_SUB, ROWS_PER_SUB)],
                    out_hbm.at[cid].at[pl.ds(sid * ROWS_PER_SUB, ROWS_PER_SUB)])


def _sc_deg(idst):
    mesh = plsc.VectorSubcoreMesh(core_axis_name="c", subcore_axis_name="s")
    return pl.kernel(
        _deg_body,
        out_type=jax.ShapeDtypeStruct((2, NPAD), jnp.float32),
        mesh=mesh,
        scratch_types=[
            pltpu.VMEM((CH, K), jnp.int32),
            pltpu.VMEM((K,), jnp.float32),
            pltpu.VMEM((128,), jnp.float32),
            pltpu.VMEM_SHARED((NPAD,), jnp.float32),
            pltpu.SemaphoreType.DMA,
        ],
    )(idst)


BR = 512
GRID_R = NPAD // BR


def _y1_body(degp_ref, x_ref, wg1_ref, y1_ref, dinv_ref):
    deg = degp_ref[0] + degp_ref[1] + 1.0
    dinv = lax.rsqrt(deg)
    dinv_ref[...] = dinv[:, None]
    y1_ref[...] = lax.dot_general(
        x_ref[...], wg1_ref[...], (((1,), (0,)), ((), ())),
        preferred_element_type=jnp.float32) * dinv[:, None]


def _y1_kernel(degp, x, W_g1):
    return pl.pallas_call(
        _y1_body,
        grid=(GRID_R,),
        in_specs=[
            pl.BlockSpec((2, BR), lambda i: (0, i)),
            pl.BlockSpec((BR, 128), lambda i: (i, 0)),
            pl.BlockSpec((128, 128), lambda i: (0, 0)),
        ],
        out_specs=[
            pl.BlockSpec((BR, 128), lambda i: (i, 0)),
            pl.BlockSpec((BR, 1), lambda i: (i, 0)),
        ],
        out_shape=[
            jax.ShapeDtypeStruct((N, 128), jnp.float32),
            jax.ShapeDtypeStruct((N, 1), jnp.float32),
        ],
    )(degp, x, W_g1)


def _y2_body(aggp_ref, y1_ref, dinv_ref, bg1_ref, wg2_ref, y2p_ref):
    agg = aggp_ref[0] + aggp_ref[1] + y1_ref[...]
    z1 = jax.nn.relu(agg * dinv_ref[...] + bg1_ref[...])
    y2 = lax.dot_general(z1, wg2_ref[...], (((1,), (0,)), ((), ())),
                         preferred_element_type=jnp.float32) * dinv_ref[...]
    y2p_ref[:, :64] = y2
    y2p_ref[:, 64:] = jnp.zeros_like(y2)


def _y2_kernel(aggp1, y1, dinv, b_g1, W_g2):
    return pl.pallas_call(
        _y2_body,
        grid=(GRID_R,),
        in_specs=[
            pl.BlockSpec((2, BR, 128), lambda i: (0, i, 0)),
            pl.BlockSpec((BR, 128), lambda i: (i, 0)),
            pl.BlockSpec((BR, 1), lambda i: (i, 0)),
            pl.BlockSpec((1, 128), lambda i: (0, 0)),
            pl.BlockSpec((128, 64), lambda i: (0, 0)),
        ],
        out_specs=pl.BlockSpec((BR, 128), lambda i: (i, 0)),
        out_shape=jax.ShapeDtypeStruct((NPAD, 128), jnp.float32),
    )(aggp1, y1, dinv, b_g1.reshape(1, 128), W_g2)


def _dec_body(aggp_ref, y2p_ref, dinv_ref, batch_ref, bg2_ref,
              wd1_ref, bd1_ref, wd2_ref, bd2_ref, we_ref, be_ref,
              z_ref, xr_ref, el_ref, zg_ref):
    i = pl.program_id(0)
    y2 = y2p_ref[:, :64]
    agg = aggp_ref[0, :, :64] + aggp_ref[1, :, :64] + y2
    z = jax.nn.relu(agg * dinv_ref[...] + bg2_ref[...])
    z_ref[...] = z
    h = jax.nn.relu(lax.dot_general(z, wd1_ref[...], (((1,), (0,)), ((), ())),
                                    preferred_element_type=jnp.float32)
                    + bd1_ref[...])
    xr_ref[...] = lax.dot_general(h, wd2_ref[...], (((1,), (0,)), ((), ())),
                                  preferred_element_type=jnp.float32) + bd2_ref[...]
    el_ref[...] = lax.dot_general(z, we_ref[...], (((1,), (0,)), ((), ())),
                                  preferred_element_type=jnp.float32) + be_ref[...]

    @pl.when(i == 0)
    def _():
        zg_ref[...] = jnp.full_like(zg_ref, -jnp.inf)

    row = i * BR + lax.broadcasted_iota(jnp.int32, (BR, 1), 0)
    valid = row < N
    bvec = batch_ref[...]

    lv = jnp.minimum(BR - 1, N - 1 - i * BR)
    b0 = jnp.clip(bvec[0, 0], 0, 63)
    b1 = jnp.clip(batch_ref[pl.ds(lv, 1), :][0, 0], b0, 63)

    def seg_body(s, _):
        mask = (bvec == s) & valid
        vals = jnp.where(mask, z, -jnp.inf)
        m = jnp.max(vals, axis=0, keepdims=True)
        zg_ref[pl.ds(s, 1), :] = jnp.maximum(zg_ref[pl.ds(s, 1), :], m)
        return 0

    lax.fori_loop(b0, b1 + 1, seg_body, 0)


def _dec_kernel(aggp2, y2p, dinv, batch, b_g2, W_d1, b_d1, W_d2, b_d2, W_e, b_e):
    return pl.pallas_call(
        _dec_body,
        grid=(GRID_R,),
        in_specs=[
            pl.BlockSpec((2, BR, 128), lambda i: (0, i, 0)),
            pl.BlockSpec((BR, 128), lambda i: (i, 0)),
            pl.BlockSpec((BR, 1), lambda i: (i, 0)),
            pl.BlockSpec((BR, 1), lambda i: (i, 0)),
            pl.BlockSpec((1, 64), lambda i: (0, 0)),
            pl.BlockSpec((64, 128), lambda i: (0, 0)),
            pl.BlockSpec((1, 128), lambda i: (0, 0)),
            pl.BlockSpec((128, 128), lambda i: (0, 0)),
            pl.BlockSpec((1, 128), lambda i: (0, 0)),
            pl.BlockSpec((64, 64), lambda i: (0, 0)),
            pl.BlockSpec((1, 64), lambda i: (0, 0)),
        ],
        out_specs=[
            pl.BlockSpec((BR, 64), lambda i: (i, 0)),
            pl.BlockSpec((BR, 128), lambda i: (i, 0)),
            pl.BlockSpec((BR, 64), lambda i: (i, 0)),
            pl.BlockSpec((64, 64), lambda i: (0, 0)),
        ],
        out_shape=[
            jax.ShapeDtypeStruct((N, 64), jnp.float32),
            jax.ShapeDtypeStruct((N, 128), jnp.float32),
            jax.ShapeDtypeStruct((N, 64), jnp.float32),
            jax.ShapeDtypeStruct((64, 64), jnp.float32),
        ],
    )(aggp2, y2p, dinv, batch, b_g2.reshape(1, 64), W_d1, b_d1.reshape(1, 128),
      W_d2, b_d2.reshape(1, 128), W_e, b_e.reshape(1, 64))


def _adj_body(l_ref, lt_ref, out_ref):
    acc = lax.dot_general(l_ref[...], lt_ref[...], (((1,), (0,)), ((), ())),
                          preferred_element_type=jnp.float32)
    out_ref[...] = jax.nn.sigmoid(acc)


def _adj_recon(el):
    elt = el.T
    n = el.shape[0]
    grid = (pl.cdiv(n, BM), pl.cdiv(n, BN))
    return pl.pallas_call(
        _adj_body,
        grid=grid,
        in_specs=[
            pl.BlockSpec((BM, el.shape[1]), lambda i, j: (i, 0)),
            pl.BlockSpec((el.shape[1], BN), lambda i, j: (0, j)),
        ],
        out_specs=pl.BlockSpec((BM, BN), lambda i, j: (i, j)),
        out_shape=jax.ShapeDtypeStruct((n, n), jnp.float32),
    )(el, elt)


def kernel(x, edge_index, batch, W_g1, b_g1, W_g2, b_g2, W_d1, b_d1, W_d2, b_d2, W_e, b_e):
    src = edge_index[0].astype(jnp.int32)
    dst = edge_index[1].astype(jnp.int32)
    isrc = _pad_idx(src, 0)
    idst = _pad_idx(dst, TRASH)

    degp = _sc_deg(idst)
    y1, dinv = _y1_kernel(degp, x, W_g1)
    aggp1 = _sc_agg(y1, isrc, idst, 128)
    y2p = _y2_kernel(aggp1, y1, dinv, b_g1, W_g2)
    aggp2 = _sc_agg(y2p, isrc, idst, 128)
    z, x_recon, edge_logits, z_g = _dec_kernel(
        aggp2, y2p, dinv, batch.astype(jnp.int32).reshape(N, 1),
        b_g2, W_d1, b_d1, W_d2, b_d2, W_e, b_e)
    adj_recon = _adj_recon(edge_logits)
    return (z, z_g, x_recon, adj_recon)


# speedup vs baseline: 1.1506x; 1.0091x over previous
import functools

import jax
import jax.numpy as jnp
from jax import lax
from jax.experimental import pallas as pl
from jax.experimental.pallas import tpu as pltpu
from jax.experimental.pallas import tpu_sc as plsc

N = 10000
E = 160000
NW = 32
NSUB = 16
EPW = E // NW
K = 128
CH = (EPW + K - 1) // K
EPW_PAD = CH * K
NPAD = 10240
ROWS_PER_SUB = NPAD // NSUB
TRASH = NPAD - 8

BM = 2048
BN = 2048


def _pad_idx(idx, fill):
    r = idx.reshape(NW, EPW)
    pad = jnp.full((NW, EPW_PAD - EPW), fill, dtype=idx.dtype)
    return jnp.concatenate([r, pad], axis=1).reshape(NW, CH, K).astype(jnp.int32)


def _agg_body(y_hbm, isrc_hbm, idst_hbm, out_hbm,
              isrc_v, idst_v, rows_v, zbuf_v, acc_sh, sem_g, sem_s):
    cid = lax.axis_index("c")
    sid = lax.axis_index("s")
    w = cid * NSUB + sid
    D = zbuf_v.shape[1]

    for r in range(16):
        for c in range(D // 16):
            zbuf_v[r, pl.ds(c * 16, 16)] = jnp.zeros((16,), jnp.float32)
    zd = []
    for t in range(ROWS_PER_SUB // 16):
        zd.append(pltpu.async_copy(
            zbuf_v, acc_sh.at[pl.ds(sid * ROWS_PER_SUB + t * 16, 16)], sem_g))
    for d in zd:
        d.wait()

    pltpu.sync_copy(isrc_hbm.at[w], isrc_v)
    pltpu.sync_copy(idst_hbm.at[w], idst_v)
    plsc.subcore_barrier()

    NB = 2

    def fire_gather(j, b):
        pltpu.async_copy(y_hbm.at[isrc_v.at[j]], rows_v.at[b], sem_g)

    def wait_gather(j, b):
        pltpu.make_async_copy(y_hbm.at[isrc_v.at[j]], rows_v.at[b], sem_g).wait()

    def fire_scatter(j, b):
        pltpu.async_copy(rows_v.at[b], acc_sh.at[idst_v.at[j]], sem_s, add=True)

    def wait_scatter(j, b):
        pltpu.make_async_copy(rows_v.at[b], acc_sh.at[idst_v.at[j]], sem_s).wait()

    for j0 in range(NB - 1):
        fire_gather(j0, j0)

    def chunk(j, k):
        wait_gather(j, k)
        fire_scatter(j, k)

        @pl.when(j >= 1)
        def _():
            wait_scatter(j - 1, (k - 1) % NB)

        @pl.when(j + NB - 1 < CH)
        def _():
            fire_gather(j + NB - 1, (k - 1) % NB)

    def body(it, _):
        for k in range(NB):
            chunk(NB * it + k, k)
        return 0

    lax.fori_loop(0, CH // NB, body, 0)
    wait_scatter(CH - 1, (CH - 1) % NB)
    plsc.subcore_barrier()

    pltpu.sync_copy(acc_sh.at[pl.ds(sid * ROWS_PER_SUB, ROWS_PER_SUB)],
                    out_hbm.at[cid].at[pl.ds(sid * ROWS_PER_SUB, ROWS_PER_SUB)])


def _sc_agg(y, isrc, idst, D):
    mesh = plsc.VectorSubcoreMesh(core_axis_name="c", subcore_axis_name="s")
    return pl.kernel(
        _agg_body,
        out_type=jax.ShapeDtypeStruct((2, NPAD, D), jnp.float32),
        mesh=mesh,
        scratch_types=[
            pltpu.VMEM((CH, K), jnp.int32),
            pltpu.VMEM((CH, K), jnp.int32),
            pltpu.VMEM((2, K, D), jnp.float32),
            pltpu.VMEM((16, D), jnp.float32),
            pltpu.VMEM_SHARED((NPAD, D), jnp.float32),
            pltpu.SemaphoreType.DMA,
            pltpu.SemaphoreType.DMA,
        ],
    )(y, isrc, idst)


def _deg_body(idst_hbm, out_hbm, idst_v, ones_v, zbuf_v, acc_sh, sem_g):
    cid = lax.axis_index("c")
    sid = lax.axis_index("s")
    w = cid * NSUB + sid

    for c in range(8):
        zbuf_v[pl.ds(c * 16, 16)] = jnp.zeros((16,), jnp.float32)
        ones_v[pl.ds(c * 16, 16)] = jnp.ones((16,), jnp.float32)
    zd = []
    for t in range(ROWS_PER_SUB // 128):
        zd.append(pltpu.async_copy(
            zbuf_v, acc_sh.at[pl.ds(sid * ROWS_PER_SUB + t * 128, 128)], sem_g))
    for d in zd:
        d.wait()

    pltpu.sync_copy(idst_hbm.at[w], idst_v)
    plsc.subcore_barrier()

    def body(j, _):
        pltpu.sync_copy(ones_v, acc_sh.at[idst_v.at[j]], add=True)
        return 0

    lax.fori_loop(0, CH, body, 0)
    plsc.subcore_barrier()

    pltpu.sync_copy(acc_sh.at[pl.ds(sid * ROWS_PER_SUB, ROWS_PER_SUB)],
                    out_hbm.at[cid].at[pl.ds(sid * ROWS_PER_SUB, ROWS_PER_SUB)])


def _sc_deg(idst):
    mesh = plsc.VectorSubcoreMesh(core_axis_name="c", subcore_axis_name="s")
    return pl.kernel(
        _deg_body,
        out_type=jax.ShapeDtypeStruct((2, NPAD), jnp.float32),
        mesh=mesh,
        scratch_types=[
            pltpu.VMEM((CH, K), jnp.int32),
            pltpu.VMEM((K,), jnp.float32),
            pltpu.VMEM((128,), jnp.float32),
            pltpu.VMEM_SHARED((NPAD,), jnp.float32),
            pltpu.SemaphoreType.DMA,
        ],
    )(idst)


BR = 512
GRID_R = NPAD // BR


def _y1_body(degp_ref, x_ref, wg1_ref, y1_ref, dinv_ref):
    deg = degp_ref[0] + degp_ref[1] + 1.0
    dinv = lax.rsqrt(deg)
    dinv_ref[...] = dinv[:, None]
    y1_ref[...] = lax.dot_general(
        x_ref[...], wg1_ref[...], (((1,), (0,)), ((), ())),
        preferred_element_type=jnp.float32) * dinv[:, None]


def _y1_kernel(degp, x, W_g1):
    return pl.pallas_call(
        _y1_body,
        grid=(GRID_R,),
        in_specs=[
            pl.BlockSpec((2, BR), lambda i: (0, i)),
            pl.BlockSpec((BR, 128), lambda i: (i, 0)),
            pl.BlockSpec((128, 128), lambda i: (0, 0)),
        ],
        out_specs=[
            pl.BlockSpec((BR, 128), lambda i: (i, 0)),
            pl.BlockSpec((BR, 1), lambda i: (i, 0)),
        ],
        out_shape=[
            jax.ShapeDtypeStruct((N, 128), jnp.float32),
            jax.ShapeDtypeStruct((N, 1), jnp.float32),
        ],
    )(degp, x, W_g1)


def _y2_body(aggp_ref, y1_ref, dinv_ref, bg1_ref, wg2_ref, y2p_ref):
    agg = aggp_ref[0] + aggp_ref[1] + y1_ref[...]
    z1 = jax.nn.relu(agg * dinv_ref[...] + bg1_ref[...])
    y2 = lax.dot_general(z1, wg2_ref[...], (((1,), (0,)), ((), ())),
                         preferred_element_type=jnp.float32) * dinv_ref[...]
    y2p_ref[:, :64] = y2
    y2p_ref[:, 64:] = jnp.zeros_like(y2)


def _y2_kernel(aggp1, y1, dinv, b_g1, W_g2):
    return pl.pallas_call(
        _y2_body,
        grid=(GRID_R,),
        in_specs=[
            pl.BlockSpec((2, BR, 128), lambda i: (0, i, 0)),
            pl.BlockSpec((BR, 128), lambda i: (i, 0)),
            pl.BlockSpec((BR, 1), lambda i: (i, 0)),
            pl.BlockSpec((1, 128), lambda i: (0, 0)),
            pl.BlockSpec((128, 64), lambda i: (0, 0)),
        ],
        out_specs=pl.BlockSpec((BR, 128), lambda i: (i, 0)),
        out_shape=jax.ShapeDtypeStruct((NPAD, 128), jnp.float32),
    )(aggp1, y1, dinv, b_g1.reshape(1, 128), W_g2)


def _dec_body(aggp_ref, y2p_ref, dinv_ref, batch_ref, bg2_ref,
              wd1_ref, bd1_ref, wd2_ref, bd2_ref, we_ref, be_ref,
              z_ref, xr_ref, el_ref, zg_ref):
    i = pl.program_id(0)
    y2 = y2p_ref[:, :64]
    agg = aggp_ref[0, :, :64] + aggp_ref[1, :, :64] + y2
    z = jax.nn.relu(agg * dinv_ref[...] + bg2_ref[...])
    z_ref[...] = z
    h = jax.nn.relu(lax.dot_general(z, wd1_ref[...], (((1,), (0,)), ((), ())),
                                    preferred_element_type=jnp.float32)
                    + bd1_ref[...])
    xr_ref[...] = lax.dot_general(h, wd2_ref[...], (((1,), (0,)), ((), ())),
                                  preferred_element_type=jnp.float32) + bd2_ref[...]
    el_ref[...] = lax.dot_general(z, we_ref[...], (((1,), (0,)), ((), ())),
                                  preferred_element_type=jnp.float32) + be_ref[...]

    @pl.when(i == 0)
    def _():
        zg_ref[...] = jnp.full_like(zg_ref, -jnp.inf)

    row = i * BR + lax.broadcasted_iota(jnp.int32, (BR, 1), 0)
    valid = row < N
    bvec = batch_ref[...]

    lv = jnp.minimum(BR - 1, N - 1 - i * BR)
    b0 = jnp.clip(bvec[0, 0], 0, 63)
    b1 = jnp.clip(batch_ref[pl.ds(lv, 1), :][0, 0], b0, 63)

    def seg_body(s, _):
        mask = (bvec == s) & valid
        vals = jnp.where(mask, z, -jnp.inf)
        m = jnp.max(vals, axis=0, keepdims=True)
        zg_ref[pl.ds(s, 1), :] = jnp.maximum(zg_ref[pl.ds(s, 1), :], m)
        return 0

    lax.fori_loop(b0, b1 + 1, seg_body, 0)


def _dec_kernel(aggp2, y2p, dinv, batch, b_g2, W_d1, b_d1, W_d2, b_d2, W_e, b_e):
    return pl.pallas_call(
        _dec_body,
        grid=(GRID_R,),
        in_specs=[
            pl.BlockSpec((2, BR, 128), lambda i: (0, i, 0)),
            pl.BlockSpec((BR, 128), lambda i: (i, 0)),
            pl.BlockSpec((BR, 1), lambda i: (i, 0)),
            pl.BlockSpec((BR, 1), lambda i: (i, 0)),
            pl.BlockSpec((1, 64), lambda i: (0, 0)),
            pl.BlockSpec((64, 128), lambda i: (0, 0)),
            pl.BlockSpec((1, 128), lambda i: (0, 0)),
            pl.BlockSpec((128, 128), lambda i: (0, 0)),
            pl.BlockSpec((1, 128), lambda i: (0, 0)),
            pl.BlockSpec((64, 64), lambda i: (0, 0)),
            pl.BlockSpec((1, 64), lambda i: (0, 0)),
        ],
        out_specs=[
            pl.BlockSpec((BR, 64), lambda i: (i, 0)),
            pl.BlockSpec((BR, 128), lambda i: (i, 0)),
            pl.BlockSpec((BR, 64), lambda i: (i, 0)),
            pl.BlockSpec((64, 64), lambda i: (0, 0)),
        ],
        out_shape=[
            jax.ShapeDtypeStruct((N, 64), jnp.float32),
            jax.ShapeDtypeStruct((N, 128), jnp.float32),
            jax.ShapeDtypeStruct((N, 64), jnp.float32),
            jax.ShapeDtypeStruct((64, 64), jnp.float32),
        ],
    )(aggp2, y2p, dinv, batch, b_g2.reshape(1, 64), W_d1, b_d1.reshape(1, 128),
      W_d2, b_d2.reshape(1, 128), W_e, b_e.reshape(1, 64))


def _adj_body(l_ref, lt_ref, out_ref):
    acc = lax.dot_general(l_ref[...], lt_ref[...], (((1,), (0,)), ((), ())),
                          preferred_element_type=jnp.float32)
    out_ref[...] = jax.nn.sigmoid(acc)


def _adj_recon(el):
    elt = el.T
    n = el.shape[0]
    grid = (pl.cdiv(n, BM), pl.cdiv(n, BN))
    return pl.pallas_call(
        _adj_body,
        grid=grid,
        in_specs=[
            pl.BlockSpec((BM, el.shape[1]), lambda i, j: (i, 0)),
            pl.BlockSpec((el.shape[1], BN), lambda i, j: (0, j)),
        ],
        out_specs=pl.BlockSpec((BM, BN), lambda i, j: (i, j)),
        out_shape=jax.ShapeDtypeStruct((n, n), jnp.float32),
    )(el, elt)


def kernel(x, edge_index, batch, W_g1, b_g1, W_g2, b_g2, W_d1, b_d1, W_d2, b_d2, W_e, b_e):
    src = edge_index[0].astype(jnp.int32)
    dst = edge_index[1].astype(jnp.int32)
    isrc = _pad_idx(src, 0)
    idst = _pad_idx(dst, TRASH)

    degp = _sc_deg(idst)
    y1, dinv = _y1_kernel(degp, x, W_g1)
    aggp1 = _sc_agg(y1, isrc, idst, 128)
    y2p = _y2_kernel(aggp1, y1, dinv, b_g1, W_g2)
    aggp2 = _sc_agg(y2p, isrc, idst, 128)
    z, x_recon, edge_logits, z_g = _dec_kernel(
        aggp2, y2p, dinv, batch.astype(jnp.int32).reshape(N, 1),
        b_g2, W_d1, b_d1, W_d2, b_d2, W_e, b_e)
    adj_recon = _adj_recon(edge_logits)
    return (z, z_g, x_recon, adj_recon)
